# baseline (speedup 1.0000x reference)
import functools

import jax
import jax.numpy as jnp
from jax import lax
from jax.experimental import pallas as pl
from jax.experimental.pallas import tpu as pltpu
from jax.experimental.pallas import tpu_sc as plsc

K_TOPK = 1024
_B, _H, _S, _D = 4, 16, 4096, 2048
_NEG = float(jnp.finfo(jnp.float32).min)

_info = plsc.get_sparse_core_info()
_NC, _NS, _L = _info.num_cores, _info.num_subcores, _info.num_lanes
_NW = _NC * _NS

_ROWS_TOTAL = _B * K_TOPK
_ROWS_PER_W = _ROWS_TOTAL // _NW
_CHUNK = 8
_NCHUNK = _ROWS_PER_W // _CHUNK
_T = _S // _L


def _iota():
    return lax.iota(jnp.int32, _L)


def _sort_batch(b, attn_hbm, mask_hbm, sbuf, maskb, keysA, keysB, valsA,
                valsB, hist, totals, topv, sem):
    with jax.named_scope("stage_attn"):
        cp = pltpu.async_copy(attn_hbm.at[b], sbuf, sem)
        pltpu.sync_copy(mask_hbm.at[b], maskb)
        cp.wait()

    with jax.named_scope("headsum"):
        @plsc.parallel_loop(0, _T, unroll=2)
        def _(t):
            base = t * _L
            x = [sbuf[h, pl.ds(base, _L)] for h in range(_H)]
            n = _H
            while n > 1:
                n //= 2
                x = [x[i] + x[i + n] for i in range(n)]
            acc = x[0]
            mv = maskb[pl.ds(base, _L)]
            sc = jnp.where(mv != 0, acc, jnp.float32(_NEG))
            sc = jnp.where(sc == 0.0, jnp.float32(0.0), sc)
            bits = lax.bitcast_convert_type(sc, jnp.int32)
            neg = bits >> 31
            key = bits ^ (jnp.bitwise_not(neg) & jnp.int32(0x7FFFFFFF))
            keysB[pl.ds(base, _L)] = key

    with jax.named_scope("transpose"):
        @plsc.parallel_loop(0, _T, unroll=2)
        def _(t):
            cvec = _iota() * _T + t
            keysA[pl.ds(t * _L, _L)] = plsc.load_gather(keysB, [cvec])
            valsA[pl.ds(t * _L, _L)] = cvec

    zeros = jnp.zeros((_L,), jnp.int32)
    ones = jnp.ones((_L,), jnp.int32)
    for p, shift in enumerate((0, 8, 16, 24)):
        kin, vin = (keysA, valsA) if p % 2 == 0 else (keysB, valsB)
        kout, vout = (keysB, valsB) if p % 2 == 0 else (keysA, valsA)

        with jax.named_scope("p%d_zero" % p):
            @plsc.parallel_loop(0, _T, unroll=4)
            def _(t):
                hist[pl.ds(t * _L, _L)] = zeros

        with jax.named_scope("p%d_hist" % p):
            @plsc.parallel_loop(0, _T, unroll=2)
            def _(t, kin=kin, shift=shift):
                k = kin[pl.ds(t * _L, _L)]
                d = (k >> shift) & jnp.int32(0xFF)
                addr = (d << 4) + _iota()
                plsc.addupdate_scatter(hist, [addr], ones)

        lane0 = _iota() == 0

        with jax.named_scope("p%d_scan" % p):
            @plsc.parallel_loop(0, _T, unroll=2)
            def _(t):
                v = hist[pl.ds(t * _L, _L)]
                c = plsc.cumsum(v)
                hist[pl.ds(t * _L, _L)] = c - v
                tv = jnp.broadcast_to(jnp.sum(v), (_L,))
                plsc.store_scatter(totals, [jnp.broadcast_to(t, (_L,))],
                                   tv, mask=lane0)

            def sc2_body(i, run):
                v = totals[pl.ds(i * _L, _L)]
                c = plsc.cumsum(v)
                totals[pl.ds(i * _L, _L)] = (c - v) + run
                return run + jnp.sum(v)

            lax.fori_loop(0, _T // _L, sc2_body, jnp.int32(0))

            @plsc.parallel_loop(0, _T, unroll=2)
            def _(t):
                carry = plsc.load_gather(totals,
                                         [jnp.broadcast_to(t, (_L,))])
                hist[pl.ds(t * _L, _L)] = hist[pl.ds(t * _L, _L)] + carry

        _W = 8

        def rp_body(g, _, kin=kin, vin=vin, kout=kout, vout=vout,
                    shift=shift):
            ks, vs, ds, addrs = [], [], [], []
            for u in range(_W):
                t = g * _W + u
                k = kin[pl.ds(t * _L, _L)]
                v = vin[pl.ds(t * _L, _L)]
                d = (k >> shift) & jnp.int32(0xFF)
                ks.append(k)
                vs.append(v)
                ds.append(d)
                addrs.append((d << 4) + _iota())
            raw = [plsc.load_gather(hist, [a]) for a in addrs]
            pos = []
            for u in range(_W):
                pu = raw[u]
                for q in range(u):
                    pu = pu + jnp.where(ds[q] == ds[u], jnp.int32(1),
                                        jnp.int32(0))
                pos.append(pu)
            for u in range(_W):
                plsc.store_scatter(hist, [addrs[u]], pos[u] + 1)
            for u in range(_W):
                phys = ((pos[u] & jnp.int32(0xFF)) << 4) | (pos[u] >> 8)
                plsc.store_scatter(kout, [phys], ks[u])
                plsc.store_scatter(vout, [phys], vs[u])
            return 0

        with jax.named_scope("p%d_permute" % p):
            lax.fori_loop(0, _T // _W, rp_body, 0, unroll=1)

    @plsc.parallel_loop(0, K_TOPK // _L, unroll=2)
    def _(r):
        cvec = r * _L + _iota()
        phys = ((cvec & jnp.int32(0xFF)) << 4) | (cvec >> 8)
        idxs = plsc.load_gather(valsA, [phys])
        topv[pl.ds(r * _L, _L)] = idxs + b * _S


@functools.partial(
    pl.kernel,
    mesh=plsc.VectorSubcoreMesh(core_axis_name="c", subcore_axis_name="s"),
    compiler_params=pltpu.CompilerParams(needs_layout_passes=False),
    out_type=jax.ShapeDtypeStruct((_ROWS_TOTAL, _D), jnp.float32),
    scratch_types=[
        pltpu.VMEM((_H, _S), jnp.float32),
        pltpu.VMEM((_S,), jnp.int32),
        pltpu.VMEM((_S,), jnp.int32),
        pltpu.VMEM((_S,), jnp.int32),
        pltpu.VMEM((_S,), jnp.int32),
        pltpu.VMEM((_S,), jnp.int32),
        pltpu.VMEM((_S,), jnp.int32),
        pltpu.VMEM((_T,), jnp.int32),
        pltpu.VMEM((K_TOPK,), jnp.int32),
        pltpu.VMEM((_ROWS_PER_W,), jnp.int32),
        pltpu.VMEM((_CHUNK, _D), jnp.float32),
        pltpu.VMEM((_CHUNK, _D), jnp.float32),
        pltpu.VMEM_SHARED((2, K_TOPK), jnp.int32),
        pltpu.SemaphoreType.DMA,
        pltpu.SemaphoreType.DMA,
        pltpu.SemaphoreType.DMA,
        pltpu.SemaphoreType.DMA,
    ],
)
def _sc_topk_gather(attn_hbm, mask_hbm, hid_hbm, out_hbm, sbuf, maskb,
                    keysA, keysB, valsA, valsB, hist, totals, topv, idx_v,
                    rows_v0, rows_v1, shared, sem0, sem1, sem2, sem3):
    c = lax.axis_index("c")
    s = lax.axis_index("s")

    @pl.when(s < 2)
    def _():
        b = 2 * c + s
        _sort_batch(b, attn_hbm, mask_hbm, sbuf, maskb, keysA, keysB,
                    valsA, valsB, hist, totals, topv, sem0)
        pltpu.sync_copy(topv, shared.at[s])

    plsc.subcore_barrier()

    b_loc = s // 8
    pltpu.sync_copy(shared.at[b_loc, pl.ds((s % 8) * _ROWS_PER_W,
                                           _ROWS_PER_W)], idx_v)
    out_base = 2 * c * K_TOPK + s * _ROWS_PER_W
    bufs = (rows_v0, rows_v1)
    rsems = (sem0, sem1)
    wsems = (sem2, sem3)

    def _gath(j):
        p = j % 2
        return pltpu.async_copy(
            hid_hbm.at[idx_v.at[pl.ds(j * _CHUNK, _CHUNK)]], bufs[p],
            rsems[p])

    _gscope = jax.named_scope("rowgather")
    _gscope.__enter__()
    g = [None] * _NCHUNK
    w = [None] * _NCHUNK
    g[0] = _gath(0)
    for j in range(_NCHUNK):
        p = j % 2
        if j + 1 < _NCHUNK:
            if j - 1 >= 0:
                w[j - 1].wait()
            g[j + 1] = _gath(j + 1)
        g[j].wait()
        w[j] = pltpu.async_copy(
            bufs[p], out_hbm.at[pl.ds(out_base + j * _CHUNK, _CHUNK)],
            wsems[p])
    w[_NCHUNK - 2].wait()
    w[_NCHUNK - 1].wait()
    _gscope.__exit__(None, None, None)


def kernel(hidden_states, latent_states, attention_mask, rotary_pos_embed,
           attn_scores):
    B, S, D = hidden_states.shape
    mask_i32 = attention_mask.astype(jnp.int32)
    hid_flat = hidden_states.reshape(B * S, D)
    out = _sc_topk_gather(attn_scores, mask_i32, hid_flat)
    return out.reshape(B, K_TOPK, D)


# speedup vs baseline: 1.0501x; 1.0501x over previous
import functools

import jax
import jax.numpy as jnp
from jax import lax
from jax.experimental import pallas as pl
from jax.experimental.pallas import tpu as pltpu
from jax.experimental.pallas import tpu_sc as plsc

K_TOPK = 1024
_B, _H, _S, _D = 4, 16, 4096, 2048
_NEG = float(jnp.finfo(jnp.float32).min)

_info = plsc.get_sparse_core_info()
_NC, _NS, _L = _info.num_cores, _info.num_subcores, _info.num_lanes
_NW = _NC * _NS

_ROWS_TOTAL = _B * K_TOPK
_ROWS_PER_W = _ROWS_TOTAL // _NW
_CHUNK = 8
_NCHUNK = _ROWS_PER_W // _CHUNK
_T = _S // _L


def _iota():
    return lax.iota(jnp.int32, _L)


def _stage_keys(c, s, attn_hbm, mask_hbm, hb, mb, kslice, shared_keys,
                sem):
    cps = []
    for b_loc in range(2):
        b = 2 * c + b_loc
        cps.append(pltpu.async_copy(
            attn_hbm.at[b, :, pl.ds(s * 256, 256)], hb.at[b_loc], sem))
        pltpu.sync_copy(mask_hbm.at[b, pl.ds(s * 256, 256)], mb.at[b_loc])
    for cp in cps:
        cp.wait()
    for b_loc in range(2):
        @plsc.parallel_loop(0, 256 // _L, unroll=2)
        def _(t, b_loc=b_loc):
            base = t * _L
            x = [hb[b_loc, h, pl.ds(base, _L)] for h in range(_H)]
            n = _H
            while n > 1:
                n //= 2
                x = [x[i] + x[i + n] for i in range(n)]
            acc = x[0]
            mv = mb[b_loc, pl.ds(base, _L)]
            sc = jnp.where(mv != 0, acc, jnp.float32(_NEG))
            sc = jnp.where(sc == 0.0, jnp.float32(0.0), sc)
            bits = lax.bitcast_convert_type(sc, jnp.int32)
            neg = bits >> 31
            key = bits ^ (jnp.bitwise_not(neg) & jnp.int32(0x7FFFFFFF))
            kslice[b_loc, pl.ds(base, _L)] = key
        pltpu.sync_copy(kslice.at[b_loc],
                        shared_keys.at[b_loc, pl.ds(s * 256, 256)])


def _sort_batch(s, row0, keysA, keysB, valsA, valsB, hist, totals, topv,
                shared_keys):
    with jax.named_scope("keyfetch"):
        pltpu.sync_copy(shared_keys.at[s], keysB)

    with jax.named_scope("transpose"):
        @plsc.parallel_loop(0, _T, unroll=2)
        def _(t):
            cvec = _iota() * _T + t
            keysA[pl.ds(t * _L, _L)] = plsc.load_gather(keysB, [cvec])
            valsA[pl.ds(t * _L, _L)] = cvec

    zeros = jnp.zeros((_L,), jnp.int32)
    ones = jnp.ones((_L,), jnp.int32)
    for p, shift in enumerate((0, 8, 16, 24)):
        kin, vin = (keysA, valsA) if p % 2 == 0 else (keysB, valsB)
        kout, vout = (keysB, valsB) if p % 2 == 0 else (keysA, valsA)

        with jax.named_scope("p%d_zero" % p):
            @plsc.parallel_loop(0, _T, unroll=4)
            def _(t):
                hist[pl.ds(t * _L, _L)] = zeros

        with jax.named_scope("p%d_hist" % p):
            @plsc.parallel_loop(0, _T, unroll=2)
            def _(t, kin=kin, shift=shift):
                k = kin[pl.ds(t * _L, _L)]
                d = (k >> shift) & jnp.int32(0xFF)
                addr = (d << 4) + _iota()
                plsc.addupdate_scatter(hist, [addr], ones)

        lane0 = _iota() == 0

        with jax.named_scope("p%d_scan" % p):
            @plsc.parallel_loop(0, _T, unroll=2)
            def _(t):
                v = hist[pl.ds(t * _L, _L)]
                c = plsc.cumsum(v)
                hist[pl.ds(t * _L, _L)] = c - v
                tv = jnp.broadcast_to(jnp.sum(v), (_L,))
                plsc.store_scatter(totals, [jnp.broadcast_to(t, (_L,))],
                                   tv, mask=lane0)

            def sc2_body(i, run):
                v = totals[pl.ds(i * _L, _L)]
                c = plsc.cumsum(v)
                totals[pl.ds(i * _L, _L)] = (c - v) + run
                return run + jnp.sum(v)

            lax.fori_loop(0, _T // _L, sc2_body, jnp.int32(0))

            @plsc.parallel_loop(0, _T, unroll=2)
            def _(t):
                carry = plsc.load_gather(totals,
                                         [jnp.broadcast_to(t, (_L,))])
                hist[pl.ds(t * _L, _L)] = hist[pl.ds(t * _L, _L)] + carry

        _W = 8

        def rp_body(g, _, kin=kin, vin=vin, kout=kout, vout=vout,
                    shift=shift):
            ks, vs, ds, addrs = [], [], [], []
            for u in range(_W):
                t = g * _W + u
                k = kin[pl.ds(t * _L, _L)]
                v = vin[pl.ds(t * _L, _L)]
                d = (k >> shift) & jnp.int32(0xFF)
                ks.append(k)
                vs.append(v)
                ds.append(d)
                addrs.append((d << 4) + _iota())
            raw = [plsc.load_gather(hist, [a]) for a in addrs]
            pos = []
            for u in range(_W):
                pu = raw[u]
                for q in range(u):
                    pu = pu + jnp.where(ds[q] == ds[u], jnp.int32(1),
                                        jnp.int32(0))
                pos.append(pu)
            for u in range(_W):
                plsc.store_scatter(hist, [addrs[u]], pos[u] + 1)
            for u in range(_W):
                phys = ((pos[u] & jnp.int32(0xFF)) << 4) | (pos[u] >> 8)
                plsc.store_scatter(kout, [phys], ks[u])
                plsc.store_scatter(vout, [phys], vs[u])
            return 0

        with jax.named_scope("p%d_permute" % p):
            lax.fori_loop(0, _T // _W, rp_body, 0, unroll=1)

    @plsc.parallel_loop(0, K_TOPK // _L, unroll=2)
    def _(r):
        cvec = r * _L + _iota()
        phys = ((cvec & jnp.int32(0xFF)) << 4) | (cvec >> 8)
        idxs = plsc.load_gather(valsA, [phys])
        topv[pl.ds(r * _L, _L)] = idxs + row0


@functools.partial(
    pl.kernel,
    mesh=plsc.VectorSubcoreMesh(core_axis_name="c", subcore_axis_name="s"),
    compiler_params=pltpu.CompilerParams(needs_layout_passes=False),
    out_type=jax.ShapeDtypeStruct((_ROWS_TOTAL, _D), jnp.float32),
    scratch_types=[
        pltpu.VMEM((2, _H, 256), jnp.float32),
        pltpu.VMEM((2, 256), jnp.int32),
        pltpu.VMEM((2, 256), jnp.int32),
        pltpu.VMEM((_S,), jnp.int32),
        pltpu.VMEM((_S,), jnp.int32),
        pltpu.VMEM((_S,), jnp.int32),
        pltpu.VMEM((_S,), jnp.int32),
        pltpu.VMEM((_S,), jnp.int32),
        pltpu.VMEM((_T,), jnp.int32),
        pltpu.VMEM((K_TOPK,), jnp.int32),
        pltpu.VMEM((_ROWS_PER_W,), jnp.int32),
        pltpu.VMEM((_CHUNK, _D), jnp.float32),
        pltpu.VMEM((_CHUNK, _D), jnp.float32),
        pltpu.VMEM_SHARED((2, _S), jnp.int32),
        pltpu.VMEM_SHARED((2, K_TOPK), jnp.int32),
        pltpu.SemaphoreType.DMA,
        pltpu.SemaphoreType.DMA,
        pltpu.SemaphoreType.DMA,
        pltpu.SemaphoreType.DMA,
    ],
)
def _sc_topk_gather(attn_hbm, mask_hbm, hid_hbm, out_hbm, hb, mb, kslice,
                    keysA, keysB, valsA, valsB, hist, totals, topv, idx_v,
                    rows_v0, rows_v1, shared_keys, shared, sem0, sem1,
                    sem2, sem3):
    c = lax.axis_index("c")
    s = lax.axis_index("s")

    with jax.named_scope("stage_attn"):
        _stage_keys(c, s, attn_hbm, mask_hbm, hb, mb, kslice, shared_keys,
                    sem0)
    plsc.subcore_barrier()

    @pl.when(s < 2)
    def _():
        b = 2 * c + s
        _sort_batch(s, b * _S, keysA, keysB, valsA, valsB, hist, totals,
                    topv, shared_keys)
        pltpu.sync_copy(topv, shared.at[s])

    plsc.subcore_barrier()

    b_loc = s // 8
    pltpu.sync_copy(shared.at[b_loc, pl.ds((s % 8) * _ROWS_PER_W,
                                           _ROWS_PER_W)], idx_v)
    out_base = 2 * c * K_TOPK + s * _ROWS_PER_W
    bufs = (rows_v0, rows_v1)
    rsems = (sem0, sem1)
    wsems = (sem2, sem3)

    def _gath(j):
        p = j % 2
        return pltpu.async_copy(
            hid_hbm.at[idx_v.at[pl.ds(j * _CHUNK, _CHUNK)]], bufs[p],
            rsems[p])

    _gscope = jax.named_scope("rowgather")
    _gscope.__enter__()
    g = [None] * _NCHUNK
    w = [None] * _NCHUNK
    g[0] = _gath(0)
    for j in range(_NCHUNK):
        p = j % 2
        if j + 1 < _NCHUNK:
            if j - 1 >= 0:
                w[j - 1].wait()
            g[j + 1] = _gath(j + 1)
        g[j].wait()
        w[j] = pltpu.async_copy(
            bufs[p], out_hbm.at[pl.ds(out_base + j * _CHUNK, _CHUNK)],
            wsems[p])
    w[_NCHUNK - 2].wait()
    w[_NCHUNK - 1].wait()
    _gscope.__exit__(None, None, None)


def kernel(hidden_states, latent_states, attention_mask, rotary_pos_embed,
           attn_scores):
    B, S, D = hidden_states.shape
    mask_i32 = attention_mask.astype(jnp.int32)
    hid_flat = hidden_states.reshape(B * S, D)
    out = _sc_topk_gather(attn_scores, mask_i32, hid_flat)
    return out.reshape(B, K_TOPK, D)
